# baseline (speedup 1.0000x reference)
import functools

import jax
import jax.numpy as jnp
from jax import lax
from jax.experimental import pallas as pl
from jax.experimental.pallas import tpu as pltpu
from jax.experimental.pallas import tpu_sc as plsc

N = 10000
K = 32
CIN = 128
COUT = 128
P = 15
KP_EXTENT = 1.0
EPS = 1e-5
NEG_SLOPE = 0.2

BK = 128
NPAD = 10112
GRID = NPAD // BK
ROWS = NPAD * K
NW = 32
BPW = ROWS // NW
CH = 128
NIT = BPW // CH
SL = 2


def _sc_gather_body(table_hbm, idx_hbm, out_hbm,
                    ib0, ib1, rb0, rb1, gs0, gs1, ss0, ss1):
    wid = lax.axis_index("s") * 2 + lax.axis_index("c")
    base = wid * BPW
    ibufs, rbufs = [ib0, ib1], [rb0, rb1]
    gsems, ssems = [gs0, gs1], [ss0, ss1]

    def idx_load(i):
        pltpu.sync_copy(idx_hbm.at[pl.ds(base + i * CH, CH)], ibufs[i % 2])

    def gather_start(i):
        return pltpu.async_copy(table_hbm.at[ibufs[i % 2]], rbufs[i % 2],
                                gsems[i % 2])

    def store_start(i):
        return pltpu.async_copy(rbufs[i % 2],
                                out_hbm.at[pl.ds(base + i * CH, CH)],
                                ssems[i % 2])

    idx_load(0)
    g = {0: gather_start(0)}
    s = {}
    for i in range(NIT):
        if i + 1 < NIT:
            idx_load(i + 1)
            if i >= 1:
                s[i - 1].wait()
            g[i + 1] = gather_start(i + 1)
        g[i].wait()
        s[i] = store_start(i)
    s[NIT - 2].wait()
    s[NIT - 1].wait()


def _sc_gather(table, idx_flat):
    mesh = plsc.VectorSubcoreMesh(core_axis_name="c", subcore_axis_name="s")
    f = pl.kernel(
        _sc_gather_body,
        mesh=mesh,
        out_type=jax.ShapeDtypeStruct((ROWS, 128), jnp.int32),
        scratch_types=[
            pltpu.VMEM((CH,), jnp.int32),
            pltpu.VMEM((CH,), jnp.int32),
            pltpu.VMEM((CH, 128), jnp.int32),
            pltpu.VMEM((CH, 128), jnp.int32),
            pltpu.SemaphoreType.DMA,
            pltpu.SemaphoreType.DMA,
            pltpu.SemaphoreType.DMA,
            pltpu.SemaphoreType.DMA,
        ],
    )
    return f(table, idx_flat)


GR = 16
GP = BK // GR
GROWS = GP * K
PP = 16


def _tc_main_body(g_ref, xq_ref, c_ref, reps_ref, tile_ref, mask_ref,
                  kvp_ref, out_ref, stats_ref, hf_s):
    i = pl.program_id(0)
    c1 = c_ref[0:4, :]
    c2 = c_ref[4:8, :]
    reps = reps_ref[...]
    tile = tile_ref[...]
    mask = mask_ref[...]

    for g in range(GR):
        r0 = GROWS * g
        q8 = xq_ref[GP * g:GP * (g + 1), :]
        qrep = jnp.dot(reps, q8,
                       preferred_element_type=jnp.float32)
        gi = g_ref[r0:r0 + GROWS, :]
        ua = lax.bitcast_convert_type(gi << 16, jnp.float32)
        ub = lax.bitcast_convert_type(gi & jnp.int32(-65536),
                                      jnp.float32)
        nbr = ua[:, 64:68] + ub[:, 64:68]
        rel = nbr - qrep
        sq = (jnp.dot(rel * rel, c1, preferred_element_type=jnp.float32)
              + jnp.dot(rel, c2, preferred_element_type=jnp.float32))
        w = jnp.maximum(1.0 - jnp.sqrt(jnp.maximum(sq, 0.0)) / KP_EXTENT,
                        0.0)
        wb = jnp.dot(w, tile,
                     preferred_element_type=jnp.float32) * mask
        dn = (((0,), (0,)), ((), ()))
        hg0 = lax.dot_general(wb, ua[:, 0:64], dn,
                              preferred_element_type=jnp.float32)
        hg1 = lax.dot_general(wb, ub[:, 0:64], dn,
                              preferred_element_type=jnp.float32)
        hg = jnp.concatenate([hg0, hg1], axis=1)
        hf_s[GP * g:GP * (g + 1), :] = hg.reshape(GP, PP * CIN)

    facc = jnp.dot(hf_s[...], kvp_ref[...],
                   preferred_element_type=jnp.float32)

    rows = i * BK + lax.broadcasted_iota(jnp.int32, (BK, 1), 0)
    ob = jnp.where(rows < N, facc, 0.0)
    out_ref[...] = ob

    @pl.when(i == 0)
    def _():
        stats_ref[...] = jnp.zeros_like(stats_ref)

    stats_ref[0:1, :] += jnp.sum(ob, axis=0, keepdims=True)
    stats_ref[1:2, :] += jnp.sum(ob * ob, axis=0, keepdims=True)


def _tc_main(g2, xq, cmat, reps, tile, mask, kvpad):
    return pl.pallas_call(
        _tc_main_body,
        grid=(GRID,),
        in_specs=[
            pl.BlockSpec((BK * K, 128), lambda i: (i, 0)),
            pl.BlockSpec((BK, 4), lambda i: (i, 0)),
            pl.BlockSpec((8, PP), lambda i: (0, 0)),
            pl.BlockSpec((GROWS, GP), lambda i: (0, 0)),
            pl.BlockSpec((PP, 128), lambda i: (0, 0)),
            pl.BlockSpec((GROWS, 128), lambda i: (0, 0)),
            pl.BlockSpec((PP * CIN, COUT), lambda i: (0, 0)),
        ],
        out_specs=[
            pl.BlockSpec((BK, COUT), lambda i: (i, 0)),
            pl.BlockSpec((8, COUT), lambda i: (0, 0)),
        ],
        out_shape=[
            jax.ShapeDtypeStruct((NPAD, COUT), jnp.float32),
            jax.ShapeDtypeStruct((8, COUT), jnp.float32),
        ],
        scratch_shapes=[pltpu.VMEM((BK, PP * CIN), jnp.float32)],
        compiler_params=pltpu.CompilerParams(
            dimension_semantics=("arbitrary",)),
    )(g2, xq, cmat, reps, tile, mask, kvpad)


def _tc_bn_body(x_ref, stats_ref, gam_ref, bet_ref, o_ref):
    mean = stats_ref[0:1, :] / N
    var = stats_ref[1:2, :] / N - mean * mean
    scale = gam_ref[...] * lax.rsqrt(var + EPS)
    y = (x_ref[...] - mean) * scale + bet_ref[...]
    o_ref[...] = jnp.where(y >= 0.0, y, NEG_SLOPE * y)


def _tc_bn(out_raw, stats, gamma, beta):
    return pl.pallas_call(
        _tc_bn_body,
        grid=(GRID,),
        in_specs=[
            pl.BlockSpec((BK, COUT), lambda i: (i, 0)),
            pl.BlockSpec((8, COUT), lambda i: (0, 0)),
            pl.BlockSpec((1, COUT), lambda i: (0, 0)),
            pl.BlockSpec((1, COUT), lambda i: (0, 0)),
        ],
        out_specs=pl.BlockSpec((BK, COUT), lambda i: (i, 0)),
        out_shape=jax.ShapeDtypeStruct((NPAD, COUT), jnp.float32),
    )(out_raw, stats, gamma, beta)


def kernel(feats, xyz, batch, neighbor_idx, K_points, K_values, gamma, beta):
    def pack2(a_bf16, b_bf16):
        abits = lax.bitcast_convert_type(a_bf16, jnp.uint16).astype(jnp.uint32)
        bbits = lax.bitcast_convert_type(b_bf16, jnp.uint16).astype(jnp.uint32)
        return lax.bitcast_convert_type(abits | (bbits << 16), jnp.int32)

    fb = feats.astype(jnp.bfloat16)
    xyz_hi = xyz.astype(jnp.bfloat16)
    xyz_lo = (xyz - xyz_hi.astype(jnp.float32)).astype(jnp.bfloat16)
    one = jnp.ones((N, 1), jnp.bfloat16)
    zero = jnp.zeros((N, 1), jnp.bfloat16)
    aux_a = jnp.concatenate([xyz_hi, one], axis=1)
    aux_b = jnp.concatenate([xyz_lo, zero], axis=1)
    table = jnp.zeros((N, 128), jnp.int32)
    table = (table.at[:, 0:64].set(pack2(fb[:, 0:64], fb[:, 64:128]))
             .at[:, 64:68].set(pack2(aux_a, aux_b)))
    idx_flat = jnp.pad(neighbor_idx, ((0, NPAD - N), (0, 0))).reshape(ROWS)
    g2 = _sc_gather(table, idx_flat)
    xq = jnp.pad(xyz, ((0, NPAD - N), (0, 1)))

    kp2 = jnp.sum(K_points * K_points, axis=1)
    c1 = jnp.zeros((4, PP), jnp.float32)
    c1 = c1.at[0:3, 0:P].set(1.0).at[3, 0:P].set(kp2).at[3, P:].set(4.0)
    c2 = jnp.zeros((4, PP), jnp.float32)
    c2 = c2.at[0:3, 0:P].set(-2.0 * K_points.T)
    cmat = jnp.concatenate([c1, c2], axis=0)

    reps = jnp.repeat(jnp.eye(GP, dtype=jnp.float32), K, axis=0)
    tile = jnp.tile(jnp.eye(PP, dtype=jnp.float32), (1, GP))
    mask = (jnp.arange(128)[None, :] // PP
            == jnp.arange(GROWS)[:, None] // K).astype(jnp.float32)
    kvpad = jnp.concatenate(
        [K_values, jnp.zeros((PP - P, CIN, COUT), jnp.float32)],
        axis=0).reshape(PP * CIN, COUT)

    out_raw, stats = _tc_main(g2, xq, cmat, reps, tile, mask, kvpad)
    out = _tc_bn(out_raw, stats, gamma.reshape(1, -1), beta.reshape(1, -1))
    return out[:N]


# speedup vs baseline: 1.1635x; 1.1635x over previous
import functools

import jax
import jax.numpy as jnp
from jax import lax
from jax.experimental import pallas as pl
from jax.experimental.pallas import tpu as pltpu
from jax.experimental.pallas import tpu_sc as plsc

N = 10000
K = 32
CIN = 128
COUT = 128
P = 15
KP_EXTENT = 1.0
EPS = 1e-5
NEG_SLOPE = 0.2

BK = 128
NPAD = 10112
GRID = NPAD // BK
ROWS = NPAD * K
NW = 32
BPW = ROWS // NW
CH = 128
NIT = BPW // CH
DW = 256


def _sc_gather_body(table_hbm, idx_hbm, out_hbm,
                    ib0, ib1, rb0, rb1, gs0, gs1, ss0, ss1):
    wid = lax.axis_index("s") * 2 + lax.axis_index("c")
    base = wid * BPW
    ibufs, rbufs = [ib0, ib1], [rb0, rb1]
    gsems, ssems = [gs0, gs1], [ss0, ss1]

    def idx_load(i):
        pltpu.sync_copy(idx_hbm.at[pl.ds(base + i * CH, CH)], ibufs[i % 2])

    def gather_start(i):
        return pltpu.async_copy(table_hbm.at[ibufs[i % 2]], rbufs[i % 2],
                                gsems[i % 2])

    def store_start(i):
        return pltpu.async_copy(rbufs[i % 2],
                                out_hbm.at[pl.ds(base + i * CH, CH)],
                                ssems[i % 2])

    idx_load(0)
    g = {0: gather_start(0)}
    s = {}
    for i in range(NIT):
        if i + 1 < NIT:
            idx_load(i + 1)
            if i >= 1:
                s[i - 1].wait()
            g[i + 1] = gather_start(i + 1)
        g[i].wait()
        s[i] = store_start(i)
    s[NIT - 2].wait()
    s[NIT - 1].wait()


def _sc_gather(table, idx_flat):
    mesh = plsc.VectorSubcoreMesh(core_axis_name="c", subcore_axis_name="s")
    f = pl.kernel(
        _sc_gather_body,
        mesh=mesh,
        out_type=jax.ShapeDtypeStruct((ROWS, DW), jnp.float32),
        scratch_types=[
            pltpu.VMEM((CH,), jnp.int32),
            pltpu.VMEM((CH,), jnp.int32),
            pltpu.VMEM((CH, DW), jnp.float32),
            pltpu.VMEM((CH, DW), jnp.float32),
            pltpu.SemaphoreType.DMA,
            pltpu.SemaphoreType.DMA,
            pltpu.SemaphoreType.DMA,
            pltpu.SemaphoreType.DMA,
        ],
    )
    return f(table, idx_flat)


GR = 16
GP = BK // GR
GROWS = GP * K
PP = 16


def _tc_main_body(g_ref, xq_ref, c_ref, reps_ref, tile_ref, mask_ref,
                  kvp_ref, out_ref, stats_ref, hf_s):
    i = pl.program_id(0)
    c1 = c_ref[0:4, :]
    c2 = c_ref[4:8, :]
    reps = reps_ref[...]
    tile = tile_ref[...]
    mask = mask_ref[...]

    for g in range(GR):
        r0 = GROWS * g
        q8 = xq_ref[GP * g:GP * (g + 1), :]
        qrep = jnp.dot(reps, q8,
                       preferred_element_type=jnp.float32)
        nbr = g_ref[r0:r0 + GROWS, CIN:CIN + 4]
        rel = nbr - qrep
        sq = (jnp.dot(rel * rel, c1, preferred_element_type=jnp.float32)
              + jnp.dot(rel, c2, preferred_element_type=jnp.float32))
        w = jnp.maximum(1.0 - jnp.sqrt(jnp.maximum(sq, 0.0)) / KP_EXTENT,
                        0.0)
        wb = jnp.dot(w, tile,
                     preferred_element_type=jnp.float32) * mask
        hg = lax.dot_general(wb, g_ref[r0:r0 + GROWS, 0:CIN],
                             (((0,), (0,)), ((), ())),
                             preferred_element_type=jnp.float32)
        hf_s[GP * g:GP * (g + 1), :] = hg.reshape(GP, PP * CIN)

    facc = jnp.dot(hf_s[...], kvp_ref[...],
                   preferred_element_type=jnp.float32)

    rows = i * BK + lax.broadcasted_iota(jnp.int32, (BK, 1), 0)
    ob = jnp.where(rows < N, facc, 0.0)
    out_ref[...] = ob

    @pl.when(i == 0)
    def _():
        stats_ref[...] = jnp.zeros_like(stats_ref)

    stats_ref[0:1, :] += jnp.sum(ob, axis=0, keepdims=True)
    stats_ref[1:2, :] += jnp.sum(ob * ob, axis=0, keepdims=True)


def _tc_main(g2, xq, cmat, reps, tile, mask, kvpad):
    return pl.pallas_call(
        _tc_main_body,
        grid=(GRID,),
        in_specs=[
            pl.BlockSpec((BK * K, DW), lambda i: (i, 0)),
            pl.BlockSpec((BK, 4), lambda i: (i, 0)),
            pl.BlockSpec((8, PP), lambda i: (0, 0)),
            pl.BlockSpec((GROWS, GP), lambda i: (0, 0)),
            pl.BlockSpec((PP, 128), lambda i: (0, 0)),
            pl.BlockSpec((GROWS, 128), lambda i: (0, 0)),
            pl.BlockSpec((PP * CIN, COUT), lambda i: (0, 0)),
        ],
        out_specs=[
            pl.BlockSpec((BK, COUT), lambda i: (i, 0)),
            pl.BlockSpec((8, COUT), lambda i: (0, 0)),
        ],
        out_shape=[
            jax.ShapeDtypeStruct((NPAD, COUT), jnp.float32),
            jax.ShapeDtypeStruct((8, COUT), jnp.float32),
        ],
        scratch_shapes=[pltpu.VMEM((BK, PP * CIN), jnp.float32)],
        compiler_params=pltpu.CompilerParams(
            dimension_semantics=("arbitrary",)),
    )(g2, xq, cmat, reps, tile, mask, kvpad)


def _tc_bn_body(x_ref, stats_ref, gam_ref, bet_ref, o_ref):
    mean = stats_ref[0:1, :] / N
    var = stats_ref[1:2, :] / N - mean * mean
    scale = gam_ref[...] * lax.rsqrt(var + EPS)
    y = (x_ref[...] - mean) * scale + bet_ref[...]
    o_ref[...] = jnp.where(y >= 0.0, y, NEG_SLOPE * y)


def _tc_bn(out_raw, stats, gamma, beta):
    return pl.pallas_call(
        _tc_bn_body,
        grid=(GRID,),
        in_specs=[
            pl.BlockSpec((BK, COUT), lambda i: (i, 0)),
            pl.BlockSpec((8, COUT), lambda i: (0, 0)),
            pl.BlockSpec((1, COUT), lambda i: (0, 0)),
            pl.BlockSpec((1, COUT), lambda i: (0, 0)),
        ],
        out_specs=pl.BlockSpec((BK, COUT), lambda i: (i, 0)),
        out_shape=jax.ShapeDtypeStruct((NPAD, COUT), jnp.float32),
    )(out_raw, stats, gamma, beta)


def kernel(feats, xyz, batch, neighbor_idx, K_points, K_values, gamma, beta):
    table = jnp.zeros((N, DW), jnp.float32)
    table = (table.at[:, :CIN].set(feats)
             .at[:, CIN:CIN + 3].set(xyz)
             .at[:, CIN + 3].set(1.0))
    idx_flat = jnp.pad(neighbor_idx, ((0, NPAD - N), (0, 0))).reshape(ROWS)
    g2 = _sc_gather(table, idx_flat)
    xq = jnp.pad(xyz, ((0, NPAD - N), (0, 1)))

    kp2 = jnp.sum(K_points * K_points, axis=1)
    c1 = jnp.zeros((4, PP), jnp.float32)
    c1 = c1.at[0:3, 0:P].set(1.0).at[3, 0:P].set(kp2).at[3, P:].set(4.0)
    c2 = jnp.zeros((4, PP), jnp.float32)
    c2 = c2.at[0:3, 0:P].set(-2.0 * K_points.T)
    cmat = jnp.concatenate([c1, c2], axis=0)

    reps = jnp.repeat(jnp.eye(GP, dtype=jnp.float32), K, axis=0)
    tile = jnp.tile(jnp.eye(PP, dtype=jnp.float32), (1, GP))
    mask = (jnp.arange(128)[None, :] // PP
            == jnp.arange(GROWS)[:, None] // K).astype(jnp.float32)
    kvpad = jnp.concatenate(
        [K_values, jnp.zeros((PP - P, CIN, COUT), jnp.float32)],
        axis=0).reshape(PP * CIN, COUT)

    out_raw, stats = _tc_main(g2, xq, cmat, reps, tile, mask, kvpad)
    out = _tc_bn(out_raw, stats, gamma.reshape(1, -1), beta.reshape(1, -1))
    return out[:N]
